# baseline (speedup 1.0000x reference)
import functools

import jax
import jax.numpy as jnp
from jax import lax
from jax.experimental import pallas as pl
from jax.experimental.pallas import tpu as pltpu
from jax.experimental.pallas import tpu_sc as plsc

Q = 128
V = 64
VP = V + 1
B = 16
L = 512
NLANE = 16
NCHUNK = Q // NLANE


def _softmax_body(T_ref, out_ref):
    v = pl.program_id(0)

    @pl.when(v < V)
    def _():
        X = T_ref[0]
        m = jnp.max(X, axis=-1, keepdims=True)
        e = jnp.exp(X - m)
        out_ref[...] = e / jnp.sum(e, axis=-1, keepdims=True)

    @pl.when(v == V)
    def _():
        row = lax.broadcasted_iota(jnp.int32, (Q, Q), 0)
        col = lax.broadcasted_iota(jnp.int32, (Q, Q), 1)
        out_ref[...] = jnp.where(row == col, 1.0, 0.0).astype(jnp.float32)


def _make_ptab(T_s):
    return pl.pallas_call(
        _softmax_body,
        grid=(VP,),
        out_shape=jax.ShapeDtypeStruct((VP * Q, Q), jnp.float32),
        in_specs=[
            pl.BlockSpec((1, Q, Q), lambda v: (jnp.minimum(v, V - 1), 0, 0)),
        ],
        out_specs=pl.BlockSpec((Q, Q), lambda v: (v, 0)),
    )(T_s)


def _dp_body(ptab_hbm, x_hbm, out_hbm, xrow, alpha, idxbuf, pbuf, sem):
    cid = lax.axis_index("c")
    sid = lax.axis_index("s")
    wid = sid * 2 + cid

    @pl.when(wid < B)
    def _():
        pltpu.sync_copy(x_hbm.at[wid], xrow.at[pl.ds(0, L)])

        iot = lax.iota(jnp.int32, NLANE)
        one_hot0 = jnp.where(iot == 0, 1.0, 0.0).astype(jnp.float32)
        alpha[pl.ds(0, NLANE)] = one_hot0
        zero = jnp.zeros((NLANE,), jnp.float32)
        for k in range(1, NCHUNK):
            alpha[pl.ds(k * NLANE, NLANE)] = zero

        def step(t, carry):
            sym = xrow[pl.ds(t, NLANE)][0]
            base = sym * Q
            for k in range(NCHUNK):
                idxbuf[pl.ds(k * NLANE, NLANE)] = base + k * NLANE + iot
            cp = pltpu.make_async_copy(ptab_hbm.at[idxbuf], pbuf, sem)
            cp.start()
            cp.wait()

            def inner(ci, accs):
                av = alpha[pl.ds(ci * NLANE, NLANE)]
                for u in range(NLANE):
                    a = av[u]
                    i = ci * NLANE + u
                    accs = tuple(
                        accs[k] + a * pbuf[i, pl.ds(k * NLANE, NLANE)]
                        for k in range(NCHUNK)
                    )
                return accs

            accs = lax.fori_loop(
                0, NCHUNK, inner,
                tuple(jnp.zeros((NLANE,), jnp.float32) for _ in range(NCHUNK)),
            )
            for k in range(NCHUNK):
                alpha[pl.ds(k * NLANE, NLANE)] = accs[k]
            return carry

        lax.fori_loop(0, L, step, 0)
        pltpu.sync_copy(alpha, out_hbm.at[wid])


def _run_dp(ptab, x_eff):
    mesh = plsc.VectorSubcoreMesh(core_axis_name="c", subcore_axis_name="s")
    kern = functools.partial(
        pl.kernel,
        mesh=mesh,
        out_type=jax.ShapeDtypeStruct((B, Q), jnp.float32),
        scratch_types=[
            pltpu.VMEM((L + NLANE,), jnp.int32),
            pltpu.VMEM((Q,), jnp.float32),
            pltpu.VMEM((Q,), jnp.int32),
            pltpu.VMEM((Q, Q), jnp.float32),
            pltpu.SemaphoreType.DMA,
        ],
    )(_dp_body)
    return kern(ptab, x_eff)


def _final_body(A_ref, f_ref, out_ref):
    f = f_ref[...]
    mf = jnp.max(f)
    w = jnp.exp(f - mf)
    s = jnp.sum(A_ref[...] * w, axis=-1, keepdims=True)
    out_ref[...] = jnp.log(s) + mf


def _finalize(A, f2):
    return pl.pallas_call(
        _final_body,
        out_shape=jax.ShapeDtypeStruct((B, 1), jnp.float32),
    )(A, f2)


def kernel(x, lengths, T_logits, f_logits):
    T_s = jnp.transpose(T_logits, (1, 0, 2))
    f2 = f_logits.reshape(1, Q)
    pos = jnp.arange(L, dtype=jnp.int32)[None, :]
    x_eff = jnp.where(pos < lengths[:, None], x, V).astype(jnp.int32)
    ptab = _make_ptab(T_s)
    A = _run_dp(ptab, x_eff)
    out = _finalize(A, f2)
    return out.reshape(B)


# speedup vs baseline: 1.0088x; 1.0088x over previous
import functools

import jax
import jax.numpy as jnp
from jax import lax
from jax.experimental import pallas as pl
from jax.experimental.pallas import tpu as pltpu
from jax.experimental.pallas import tpu_sc as plsc

Q = 128
V = 64
VP = V + 1
B = 16
L = 512
NLANE = 16
NCHUNK = Q // NLANE


def _softmax_body(T_ref, out_ref):
    v = pl.program_id(0)

    @pl.when(v < V)
    def _():
        X = T_ref[0]
        m = jnp.max(X, axis=-1, keepdims=True)
        e = jnp.exp(X - m)
        out_ref[0] = e / jnp.sum(e, axis=-1, keepdims=True)

    @pl.when(v == V)
    def _():
        row = lax.broadcasted_iota(jnp.int32, (Q, Q), 0)
        col = lax.broadcasted_iota(jnp.int32, (Q, Q), 1)
        out_ref[0] = jnp.where(row == col, 1.0, 0.0).astype(jnp.float32)


def _make_ptab(T_s):
    return pl.pallas_call(
        _softmax_body,
        grid=(VP,),
        out_shape=jax.ShapeDtypeStruct((VP, Q, Q), jnp.float32),
        in_specs=[
            pl.BlockSpec((1, Q, Q), lambda v: (jnp.minimum(v, V - 1), 0, 0)),
        ],
        out_specs=pl.BlockSpec((1, Q, Q), lambda v: (v, 0, 0)),
    )(T_s)


def _dp_body(ptab_hbm, x_hbm, out_hbm, xrow, alpha, pbuf, sem):
    cid = lax.axis_index("c")
    sid = lax.axis_index("s")
    wid = sid * 2 + cid

    @pl.when(wid < B)
    def _():
        pltpu.sync_copy(x_hbm.at[wid], xrow.at[pl.ds(0, L)])

        iot = lax.iota(jnp.int32, NLANE)
        one_hot0 = jnp.where(iot == 0, 1.0, 0.0).astype(jnp.float32)
        alpha[pl.ds(0, NLANE)] = one_hot0
        zero = jnp.zeros((NLANE,), jnp.float32)
        for k in range(1, NCHUNK):
            alpha[pl.ds(k * NLANE, NLANE)] = zero

        def step(t, carry):
            sym = xrow[pl.ds(t, NLANE)][0]
            cp = pltpu.make_async_copy(ptab_hbm.at[sym], pbuf, sem)
            cp.start()
            cp.wait()

            def inner(ci, accs):
                av = alpha[pl.ds(ci * NLANE, NLANE)]
                for u in range(NLANE):
                    a = av[u]
                    i = ci * NLANE + u
                    accs = tuple(
                        accs[k] + a * pbuf[i, pl.ds(k * NLANE, NLANE)]
                        for k in range(NCHUNK)
                    )
                return accs

            accs = lax.fori_loop(
                0, NCHUNK, inner,
                tuple(jnp.zeros((NLANE,), jnp.float32) for _ in range(NCHUNK)),
            )
            for k in range(NCHUNK):
                alpha[pl.ds(k * NLANE, NLANE)] = accs[k]
            return carry

        lax.fori_loop(0, L, step, 0)
        pltpu.sync_copy(alpha, out_hbm.at[wid])


def _run_dp(ptab, x_eff):
    mesh = plsc.VectorSubcoreMesh(core_axis_name="c", subcore_axis_name="s")
    kern = functools.partial(
        pl.kernel,
        mesh=mesh,
        out_type=jax.ShapeDtypeStruct((B, Q), jnp.float32),
        scratch_types=[
            pltpu.VMEM((L + NLANE,), jnp.int32),
            pltpu.VMEM((Q,), jnp.float32),
            pltpu.VMEM((Q, Q), jnp.float32),
            pltpu.SemaphoreType.DMA,
        ],
    )(_dp_body)
    return kern(ptab, x_eff)


def _final_body(A_ref, f_ref, out_ref):
    f = f_ref[...]
    mf = jnp.max(f)
    w = jnp.exp(f - mf)
    s = jnp.sum(A_ref[...] * w, axis=-1, keepdims=True)
    out_ref[...] = jnp.log(s) + mf


def _finalize(A, f2):
    return pl.pallas_call(
        _final_body,
        out_shape=jax.ShapeDtypeStruct((B, 1), jnp.float32),
    )(A, f2)


def kernel(x, lengths, T_logits, f_logits):
    T_s = jnp.transpose(T_logits, (1, 0, 2))
    f2 = f_logits.reshape(1, Q)
    pos = jnp.arange(L, dtype=jnp.int32)[None, :]
    x_eff = jnp.where(pos < lengths[:, None], x, V).astype(jnp.int32)
    ptab = _make_ptab(T_s)
    A = _run_dp(ptab, x_eff)
    out = _finalize(A, f2)
    return out.reshape(B)


# speedup vs baseline: 1.9337x; 1.9169x over previous
import functools

import jax
import jax.numpy as jnp
from jax import lax
from jax.experimental import pallas as pl
from jax.experimental.pallas import tpu as pltpu
from jax.experimental.pallas import tpu_sc as plsc

Q = 128
V = 64
VP = V + 1
B = 16
L = 512
NLANE = 16
NCHUNK = Q // NLANE


def _softmax_body(T_ref, out_ref):
    v = pl.program_id(0)

    @pl.when(v < V)
    def _():
        X = T_ref[0]
        m = jnp.max(X, axis=-1, keepdims=True)
        e = jnp.exp(X - m)
        out_ref[0] = e / jnp.sum(e, axis=-1, keepdims=True)

    @pl.when(v == V)
    def _():
        row = lax.broadcasted_iota(jnp.int32, (Q, Q), 0)
        col = lax.broadcasted_iota(jnp.int32, (Q, Q), 1)
        out_ref[0] = jnp.where(row == col, 1.0, 0.0).astype(jnp.float32)


def _make_ptab(T_s):
    return pl.pallas_call(
        _softmax_body,
        grid=(VP,),
        out_shape=jax.ShapeDtypeStruct((VP, Q, Q), jnp.float32),
        in_specs=[
            pl.BlockSpec((1, Q, Q), lambda v: (jnp.minimum(v, V - 1), 0, 0)),
        ],
        out_specs=pl.BlockSpec((1, Q, Q), lambda v: (v, 0, 0)),
    )(T_s)


HALF = Q // 2


def _dp_body(ptab_hbm, x_hbm, out_hbm, xrow, alpha, part, prt, pbufA, pbufB,
             shpart, semA, semB):
    cid = lax.axis_index("c")
    sid = lax.axis_index("s")
    seq = cid * (B // 2) + sid // 2
    half = sid % 2

    pltpu.sync_copy(x_hbm.at[seq], xrow.at[pl.ds(0, L)])
    xrow[pl.ds(L, NLANE)] = jnp.full((NLANE,), V, jnp.int32)

    iot = lax.iota(jnp.int32, NLANE)
    one_hot0 = jnp.where(iot == 0, 1.0, 0.0).astype(jnp.float32)
    alpha[pl.ds(0, NLANE)] = one_hot0
    zero = jnp.zeros((NLANE,), jnp.float32)
    for k in range(1, NCHUNK):
        alpha[pl.ds(k * NLANE, NLANE)] = zero

    def fetch(t, buf, sem):
        sym = xrow[pl.ds(t, NLANE)][0]
        pltpu.make_async_copy(ptab_hbm.at[sym * 2 + half], buf, sem).start()

    def wait(buf, sem):
        pltpu.make_async_copy(ptab_hbm.at[0], buf, sem).wait()

    def half_step(buf, slot):
        def inner(ci, accs):
            av = alpha[pl.ds(half * HALF + ci * NLANE, NLANE)]
            for u in range(NLANE):
                a = av[u]
                i = ci * NLANE + u
                accs = tuple(
                    accs[k] + a * buf[i, pl.ds(k * NLANE, NLANE)]
                    for k in range(NCHUNK)
                )
            return accs

        accs = lax.fori_loop(
            0, HALF // NLANE, inner,
            tuple(jnp.zeros((NLANE,), jnp.float32) for _ in range(NCHUNK)),
        )
        for k in range(NCHUNK):
            part[pl.ds(k * NLANE, NLANE)] = accs[k]
        pltpu.sync_copy(part, shpart.at[slot, sid])
        plsc.subcore_barrier()
        pltpu.sync_copy(shpart.at[slot, jnp.bitwise_xor(sid, 1)], prt)
        for k in range(NCHUNK):
            sl = pl.ds(k * NLANE, NLANE)
            alpha[sl] = part[sl] + prt[sl]

    fetch(0, pbufA, semA)

    def two_steps(tb, carry):
        t = 2 * tb
        fetch(t + 1, pbufB, semB)
        wait(pbufA, semA)
        half_step(pbufA, 0)
        fetch(t + 2, pbufA, semA)
        wait(pbufB, semB)
        half_step(pbufB, 1)
        return carry

    lax.fori_loop(0, L // 2, two_steps, 0)
    wait(pbufA, semA)

    @pl.when(half == 0)
    def _():
        pltpu.sync_copy(alpha, out_hbm.at[seq])


def _run_dp(ptab, x_eff):
    mesh = plsc.VectorSubcoreMesh(core_axis_name="c", subcore_axis_name="s")
    kern = functools.partial(
        pl.kernel,
        mesh=mesh,
        out_type=jax.ShapeDtypeStruct((B, Q), jnp.float32),
        scratch_types=[
            pltpu.VMEM((L + NLANE,), jnp.int32),
            pltpu.VMEM((Q,), jnp.float32),
            pltpu.VMEM((Q,), jnp.float32),
            pltpu.VMEM((Q,), jnp.float32),
            pltpu.VMEM((HALF, Q), jnp.float32),
            pltpu.VMEM((HALF, Q), jnp.float32),
            pltpu.VMEM_SHARED((2, 16, Q), jnp.float32),
            pltpu.SemaphoreType.DMA,
            pltpu.SemaphoreType.DMA,
        ],
    )(_dp_body)
    return kern(ptab.reshape(VP * 2, HALF, Q), x_eff)


def _final_body(A_ref, f_ref, out_ref):
    f = f_ref[...]
    mf = jnp.max(f)
    w = jnp.exp(f - mf)
    s = jnp.sum(A_ref[...] * w, axis=-1, keepdims=True)
    out_ref[...] = jnp.log(s) + mf


def _finalize(A, f2):
    return pl.pallas_call(
        _final_body,
        out_shape=jax.ShapeDtypeStruct((B, 1), jnp.float32),
    )(A, f2)


def kernel(x, lengths, T_logits, f_logits):
    T_s = jnp.transpose(T_logits, (1, 0, 2))
    f2 = f_logits.reshape(1, Q)
    pos = jnp.arange(L, dtype=jnp.int32)[None, :]
    x_eff = jnp.where(pos < lengths[:, None], x, V).astype(jnp.int32)
    ptab = _make_ptab(T_s)
    A = _run_dp(ptab, x_eff)
    out = _finalize(A, f2)
    return out.reshape(B)


# speedup vs baseline: 2.1462x; 1.1099x over previous
import functools

import jax
import jax.numpy as jnp
from jax import lax
from jax.experimental import pallas as pl
from jax.experimental.pallas import tpu as pltpu
from jax.experimental.pallas import tpu_sc as plsc

Q = 128
V = 64
VP = V + 1
B = 16
L = 512
NLANE = 16
NCHUNK = Q // NLANE


def _softmax_body(T_ref, out_ref):
    v = pl.program_id(0)

    @pl.when(v < V)
    def _():
        X = T_ref[0]
        m = jnp.max(X, axis=-1, keepdims=True)
        e = jnp.exp(X - m)
        out_ref[0] = e / jnp.sum(e, axis=-1, keepdims=True)

    @pl.when(v == V)
    def _():
        row = lax.broadcasted_iota(jnp.int32, (Q, Q), 0)
        col = lax.broadcasted_iota(jnp.int32, (Q, Q), 1)
        out_ref[0] = jnp.where(row == col, 1.0, 0.0).astype(jnp.float32)


def _make_ptab(T_s):
    return pl.pallas_call(
        _softmax_body,
        grid=(VP,),
        out_shape=jax.ShapeDtypeStruct((VP, Q, Q), jnp.float32),
        in_specs=[
            pl.BlockSpec((1, Q, Q), lambda v: (jnp.minimum(v, V - 1), 0, 0)),
        ],
        out_specs=pl.BlockSpec((1, Q, Q), lambda v: (v, 0, 0)),
    )(T_s)


HALF = Q // 2


def _dp_body(ptab_hbm, x_hbm, out_hbm, xrow, alpha, part, prt, pbufA, pbufB,
             shpart, semA, semB):
    cid = lax.axis_index("c")
    sid = lax.axis_index("s")
    seq = cid * (B // 2) + sid // 2
    half = sid % 2

    pltpu.sync_copy(x_hbm.at[seq], xrow.at[pl.ds(0, L)])
    padv = jnp.full((NLANE,), V, jnp.int32)
    xrow[pl.ds(L, NLANE)] = padv
    xrow[pl.ds(L + NLANE, NLANE)] = padv

    iot = lax.iota(jnp.int32, NLANE)
    one_hot0 = jnp.where(iot == 0, 1.0, 0.0).astype(jnp.float32)
    alpha[pl.ds(0, NLANE)] = one_hot0
    zero = jnp.zeros((NLANE,), jnp.float32)
    for k in range(1, NCHUNK):
        alpha[pl.ds(k * NLANE, NLANE)] = zero

    def fetch(t, buf, sem):
        sym = xrow[pl.ds(t, NLANE)][0]
        pltpu.make_async_copy(ptab_hbm.at[sym * 2 + half], buf, sem).start()

    def wait(buf, sem):
        pltpu.make_async_copy(ptab_hbm.at[0], buf, sem).wait()

    def half_step(buf, slot, tnext, nbuf, nsem):
        def inner(ci, accs):
            av = alpha[pl.ds(half * HALF + ci * NLANE, NLANE)]
            for u in range(NLANE):
                a = av[u]
                i = ci * NLANE + u
                accs = tuple(
                    accs[k] + a * buf[i, pl.ds(k * NLANE, NLANE)]
                    for k in range(NCHUNK)
                )
            return accs

        accs = lax.fori_loop(
            0, HALF // NLANE, inner,
            tuple(jnp.zeros((NLANE,), jnp.float32) for _ in range(NCHUNK)),
        )
        fetch(tnext, nbuf, nsem)
        for k in range(NCHUNK):
            part[pl.ds(k * NLANE, NLANE)] = accs[k]
        pltpu.sync_copy(part, shpart.at[slot, sid])
        plsc.subcore_barrier()
        pltpu.sync_copy(shpart.at[slot, jnp.bitwise_xor(sid, 1)], prt)
        for k2 in range(HALF // NLANE):
            sl = pl.ds(half * HALF + k2 * NLANE, NLANE)
            alpha[sl] = part[sl] + prt[sl]

    fetch(0, pbufA, semA)
    fetch(1, pbufB, semB)

    def two_steps(tb, carry):
        t = 2 * tb
        wait(pbufA, semA)
        half_step(pbufA, 0, t + 2, pbufA, semA)
        wait(pbufB, semB)
        half_step(pbufB, 1, t + 3, pbufB, semB)
        return carry

    lax.fori_loop(0, L // 2, two_steps, 0)
    wait(pbufA, semA)
    wait(pbufB, semB)

    @pl.when(half == 0)
    def _():
        for k in range(NCHUNK):
            sl = pl.ds(k * NLANE, NLANE)
            alpha[sl] = part[sl] + prt[sl]
        pltpu.sync_copy(alpha, out_hbm.at[seq])


def _run_dp(ptab, x_eff):
    mesh = plsc.VectorSubcoreMesh(core_axis_name="c", subcore_axis_name="s")
    kern = functools.partial(
        pl.kernel,
        mesh=mesh,
        out_type=jax.ShapeDtypeStruct((B, Q), jnp.float32),
        scratch_types=[
            pltpu.VMEM((L + 2 * NLANE,), jnp.int32),
            pltpu.VMEM((Q,), jnp.float32),
            pltpu.VMEM((Q,), jnp.float32),
            pltpu.VMEM((Q,), jnp.float32),
            pltpu.VMEM((HALF, Q), jnp.float32),
            pltpu.VMEM((HALF, Q), jnp.float32),
            pltpu.VMEM_SHARED((2, 16, Q), jnp.float32),
            pltpu.SemaphoreType.DMA,
            pltpu.SemaphoreType.DMA,
        ],
    )(_dp_body)
    return kern(ptab.reshape(VP * 2, HALF, Q), x_eff)


def _final_body(A_ref, f_ref, out_ref):
    f = f_ref[...]
    mf = jnp.max(f)
    w = jnp.exp(f - mf)
    s = jnp.sum(A_ref[...] * w, axis=-1, keepdims=True)
    out_ref[...] = jnp.log(s) + mf


def _finalize(A, f2):
    return pl.pallas_call(
        _final_body,
        out_shape=jax.ShapeDtypeStruct((B, 1), jnp.float32),
    )(A, f2)


def kernel(x, lengths, T_logits, f_logits):
    T_s = jnp.transpose(T_logits, (1, 0, 2))
    f2 = f_logits.reshape(1, Q)
    pos = jnp.arange(L, dtype=jnp.int32)[None, :]
    x_eff = jnp.where(pos < lengths[:, None], x, V).astype(jnp.int32)
    ptab = _make_ptab(T_s)
    A = _run_dp(ptab, x_eff)
    out = _finalize(A, f2)
    return out.reshape(B)
